# baseline (speedup 1.0000x reference)
import functools

import jax
import jax.numpy as jnp
from jax import lax
from jax.experimental import pallas as pl
from jax.experimental.pallas import tpu as pltpu
from jax.experimental.pallas import tpu_sc as plsc

NC = 2
NS = 16
L = 16
NW = NC * NS
NP = 10240
DD = 128
H1 = 256
H2 = 128
C = 128
NCH = 80
EPW = C * NCH
RPT = NP // NS
PKB = 14


def _mesh():
    return plsc.VectorSubcoreMesh(core_axis_name="c", subcore_axis_name="s")


def _fill2d(ref, rows, value):
    width = ref.shape[1]

    def body(i, carry):
        r = i // (width // L)
        k = (i % (width // L)) * L
        ref[r, pl.ds(k, L)] = jnp.full((L,), value, jnp.float32)
        return carry
    lax.fori_loop(0, rows * (width // L), body, None)


def _make_deg():
    scratch = (
        pltpu.VMEM_SHARED((NP, DD), jnp.float32),
        pltpu.VMEM((NCH, C), jnp.int32),
        pltpu.VMEM((C, DD), jnp.float32),
    )

    @functools.partial(
        pl.kernel, out_type=jax.ShapeDtypeStruct((NC, NP, DD), jnp.float32),
        mesh=_mesh(), scratch_types=scratch)
    def deg_kernel(dstw, deg_out, degacc, dst_v, ones_v):
        cc = lax.axis_index("c")
        ss = lax.axis_index("s")
        base = ss * RPT
        _fill2d(ones_v, C, 0.0)
        for q in range(RPT // C):
            pltpu.sync_copy(ones_v, degacc.at[pl.ds(base + q * C, C)])
        _fill2d(ones_v, C, 1.0)
        pltpu.sync_copy(dstw.at[cc].at[ss], dst_v)
        plsc.subcore_barrier()

        def step(j, carry):
            pltpu.sync_copy(ones_v, degacc.at[dst_v.at[j]], add=True)
            return carry
        lax.fori_loop(0, NCH, step, None)
        plsc.subcore_barrier()
        pltpu.sync_copy(degacc.at[pl.ds(base, RPT)],
                        deg_out.at[cc].at[pl.ds(base, RPT)])

    return deg_kernel


_deg_pass = _make_deg()


def _make_segsum():
    scratch = (
        pltpu.VMEM_SHARED((NP, DD), jnp.float32),
        pltpu.VMEM((NCH, C), jnp.int32),
        pltpu.VMEM((2, C), jnp.int32),
        pltpu.VMEM((2, C), jnp.int32),
        pltpu.VMEM((C, DD), jnp.float32),
        pltpu.VMEM((C, DD), jnp.float32),
        pltpu.SemaphoreType.DMA,
        pltpu.SemaphoreType.DMA,
    )

    @functools.partial(
        pl.kernel, out_type=jax.ShapeDtypeStruct((NC, NP, DD), jnp.float32),
        mesh=_mesh(), scratch_types=scratch)
    def seg(table, pkw, agg_out, acc, pk_v, sidx, didx, rows0, rows1,
            sem0, sem1):
        cc = lax.axis_index("c")
        ss = lax.axis_index("s")
        base = ss * RPT

        def unpack(j, b):
            for u in range(C // L):
                v = pk_v[j, pl.ds(u * L, L)]
                sidx[b, pl.ds(u * L, L)] = jnp.bitwise_and(v, (1 << PKB) - 1)
                didx[b, pl.ds(u * L, L)] = jnp.right_shift(v, PKB)

        _fill2d(rows0, C, 0.0)
        for q in range(RPT // C):
            pltpu.sync_copy(rows0, acc.at[pl.ds(base + q * C, C)])
        pltpu.sync_copy(pkw.at[cc].at[ss], pk_v)
        plsc.subcore_barrier()

        GS = 4
        CH = C // GS

        def fire(b, rows, sem):
            for hh in range(GS):
                pltpu.async_copy(
                    table.at[sidx.at[b].at[pl.ds(hh * CH, CH)]],
                    rows.at[pl.ds(hh * CH, CH)], sem)

        def drain(b, rows, sem):
            for hh in range(GS):
                pltpu.make_async_copy(
                    table.at[sidx.at[b].at[pl.ds(hh * CH, CH)]],
                    rows.at[pl.ds(hh * CH, CH)], sem).wait()

        unpack(0, 0)
        fire(0, rows0, sem0)
        unpack(1, 1)
        fire(1, rows1, sem1)

        def step(i, carry):
            for b, rows, sem in ((0, rows0, sem0), (1, rows1, sem1)):
                j = 2 * i + b
                drain(b, rows, sem)
                pltpu.sync_copy(rows, acc.at[didx.at[b]], add=True)
                jn = lax.rem(j + 2, NCH)
                unpack(jn, b)
                fire(b, rows, sem)
            return carry
        lax.fori_loop(0, NCH // 2, step, None)
        drain(0, rows0, sem0)
        drain(1, rows1, sem1)
        plsc.subcore_barrier()
        pltpu.sync_copy(acc.at[pl.ds(base, RPT)],
                        agg_out.at[cc].at[pl.ds(base, RPT)])

    return seg


_segsum = _make_segsum()


def _make_score():
    scratch = (
        pltpu.VMEM((NCH, C), jnp.int32),
        pltpu.VMEM((NCH, C), jnp.int32),
        pltpu.VMEM((C, DD), jnp.float32),
        pltpu.VMEM((C, DD), jnp.float32),
        pltpu.VMEM((C, DD), jnp.float32),
        pltpu.VMEM((C, DD), jnp.float32),
        pltpu.VMEM((EPW,), jnp.float32),
        pltpu.VMEM((DD,), jnp.float32),
        pltpu.VMEM((L,), jnp.float32),
        pltpu.SemaphoreType.DMA,
        pltpu.SemaphoreType.DMA,
        pltpu.SemaphoreType.DMA,
        pltpu.SemaphoreType.DMA,
    )

    @functools.partial(
        pl.kernel,
        out_type=jax.ShapeDtypeStruct((NC, NS, EPW), jnp.float32),
        mesh=_mesh(), scratch_types=scratch)
    def score(Atab, Btab, srcw, dstw, w2, bp2, out, src_v, dst_v,
              a0, a1, b0, b1, sbuf, w2v, bp2v, sA0, sA1, sB0, sB1):
        cc = lax.axis_index("c")
        ss = lax.axis_index("s")
        pltpu.sync_copy(srcw.at[cc].at[ss], src_v)
        pltpu.sync_copy(dstw.at[cc].at[ss], dst_v)
        pltpu.sync_copy(w2, w2v)
        pltpu.sync_copy(bp2, bp2v)
        GS = 2
        CH = C // GS

        def fire(tab, idx_v, j, buf, sem):
            for hh in range(GS):
                pltpu.async_copy(
                    tab.at[idx_v.at[j].at[pl.ds(hh * CH, CH)]],
                    buf.at[pl.ds(hh * CH, CH)], sem)

        def drain(tab, idx_v, j, buf, sem):
            for hh in range(GS):
                pltpu.make_async_copy(
                    tab.at[idx_v.at[j].at[pl.ds(hh * CH, CH)]],
                    buf.at[pl.ds(hh * CH, CH)], sem).wait()

        fire(Atab, src_v, 0, a0, sA0)
        fire(Btab, dst_v, 0, b0, sB0)
        fire(Atab, src_v, 1, a1, sA1)
        fire(Btab, dst_v, 1, b1, sB1)
        iota = lax.iota(jnp.int32, L)
        bp2s = bp2v[...] * (1.0 / L)
        w2regs = [w2v[pl.ds(u * L, L)] for u in range(DD // L)]
        shuf = [jnp.bitwise_xor(iota, sh) for sh in (1, 2, 4, 8)]

        def step(i, carry):
            for b, abuf, bbuf, sA, sB in ((0, a0, b0, sA0, sB0),
                                          (1, a1, b1, sA1, sB1)):
                j = 2 * i + b
                drain(Atab, src_v, j, abuf, sA)
                drain(Btab, dst_v, j, bbuf, sB)

                def group(g, carry2):
                    def edge(m, svec):
                        e = g * L + m
                        acc = bp2s
                        for u in range(DD // L):
                            va = abuf[e, pl.ds(u * L, L)]
                            vb = bbuf[e, pl.ds(u * L, L)]
                            acc = acc + jnp.maximum(va + vb, 0.0) * w2regs[u]
                        for sx in shuf:
                            acc = acc + acc.at[sx].get(
                                mode="promise_in_bounds")
                        return jnp.where(iota == m, acc, svec)
                    svec = lax.fori_loop(0, L, edge,
                                         jnp.zeros((L,), jnp.float32),
                                         unroll=4)
                    sbuf[pl.ds(j * C + g * L, L)] = svec
                    return carry2
                lax.fori_loop(0, C // L, group, None)
                jn = lax.rem(j + 2, NCH)
                fire(Atab, src_v, jn, abuf, sA)
                fire(Btab, dst_v, jn, bbuf, sB)
            return carry
        lax.fori_loop(0, NCH // 2, step, None)
        drain(Atab, src_v, 0, a0, sA0)
        drain(Btab, dst_v, 0, b0, sB0)
        drain(Atab, src_v, 1, a1, sA1)
        drain(Btab, dst_v, 1, b1, sB1)
        pltpu.sync_copy(sbuf, out.at[cc].at[ss])

    return score


_score = _make_score()

_BM = 256


def _tc1(xp, agg, deg, Ws1, Wn1, b1r, Wn2):
    def body(x_ref, agg_ref, deg_ref, ws1, wn1, b1_, wn2, h_ref, y2_ref):
        aggs = agg_ref[0] + agg_ref[1]
        dg = deg_ref[0, :, 0:1] + deg_ref[1, :, 0:1]
        mean = aggs / jnp.maximum(dg, 1.0)
        h = jnp.dot(x_ref[...], ws1[...], preferred_element_type=jnp.float32)
        h = h + jnp.dot(mean, wn1[...], preferred_element_type=jnp.float32)
        h = jnp.maximum(h + b1_[...], 0.0)
        h_ref[...] = h
        y2_ref[...] = jnp.dot(h, wn2[...], preferred_element_type=jnp.float32)

    return pl.pallas_call(
        body,
        grid=(NP // _BM,),
        in_specs=[
            pl.BlockSpec((_BM, DD), lambda i: (i, 0)),
            pl.BlockSpec((NC, _BM, DD), lambda i: (0, i, 0)),
            pl.BlockSpec((NC, _BM, DD), lambda i: (0, i, 0)),
            pl.BlockSpec((DD, H1), lambda i: (0, 0)),
            pl.BlockSpec((DD, H1), lambda i: (0, 0)),
            pl.BlockSpec((1, H1), lambda i: (0, 0)),
            pl.BlockSpec((H1, H2), lambda i: (0, 0)),
        ],
        out_specs=[
            pl.BlockSpec((_BM, H1), lambda i: (i, 0)),
            pl.BlockSpec((_BM, H2), lambda i: (i, 0)),
        ],
        out_shape=[
            jax.ShapeDtypeStruct((NP, H1), jnp.float32),
            jax.ShapeDtypeStruct((NP, H2), jnp.float32),
        ],
    )(xp, agg, deg, Ws1, Wn1, b1r, Wn2)


def _tc2(h, agg2, deg, Ws2, b2r, Wp1a, Wp1b, bp1r):
    def body(h_ref, agg_ref, deg_ref, ws2, b2_, wa, wb, bp1_, A_ref, B_ref):
        aggs = agg_ref[0] + agg_ref[1]
        dg = deg_ref[0, :, 0:1] + deg_ref[1, :, 0:1]
        mean = aggs / jnp.maximum(dg, 1.0)
        emb = jnp.dot(h_ref[...], ws2[...], preferred_element_type=jnp.float32)
        emb = emb + mean + b2_[...]
        A_ref[...] = jnp.dot(emb, wa[...], preferred_element_type=jnp.float32)
        B_ref[...] = (jnp.dot(emb, wb[...], preferred_element_type=jnp.float32)
                      + bp1_[...])

    return pl.pallas_call(
        body,
        grid=(NP // _BM,),
        in_specs=[
            pl.BlockSpec((_BM, H1), lambda i: (i, 0)),
            pl.BlockSpec((NC, _BM, H2), lambda i: (0, i, 0)),
            pl.BlockSpec((NC, _BM, DD), lambda i: (0, i, 0)),
            pl.BlockSpec((H1, H2), lambda i: (0, 0)),
            pl.BlockSpec((1, H2), lambda i: (0, 0)),
            pl.BlockSpec((H2, H2), lambda i: (0, 0)),
            pl.BlockSpec((H2, H2), lambda i: (0, 0)),
            pl.BlockSpec((1, H2), lambda i: (0, 0)),
        ],
        out_specs=[
            pl.BlockSpec((_BM, H2), lambda i: (i, 0)),
            pl.BlockSpec((_BM, H2), lambda i: (i, 0)),
        ],
        out_shape=[
            jax.ShapeDtypeStruct((NP, H2), jnp.float32),
            jax.ShapeDtypeStruct((NP, H2), jnp.float32),
        ],
    )(h, agg2, deg, Ws2, b2r, Wp1a, Wp1b, bp1r)


def kernel(x, edge_index, pos_edge_index, neg_edge_index,
           W_self1, W_neigh1, b1, W_self2, W_neigh2, b2,
           Wp1, bp1, Wp2, bp2):
    n, d = x.shape
    xp = jnp.zeros((NP, d), jnp.float32).at[:n].set(x)
    src, dst = edge_index[0], edge_index[1]
    e = src.shape[0]
    npad = NW * EPW - e
    srcf = jnp.concatenate([src, jnp.zeros((npad,), jnp.int32)])
    dstf = jnp.concatenate([dst, jnp.full((npad,), n, jnp.int32)])
    pkw = (jnp.left_shift(dstf, PKB) | srcf).reshape(NC, NS, NCH, C)
    dstw = dstf.reshape(NC, NS, NCH, C)

    deg = _deg_pass(dstw)
    agg1 = _segsum(xp, pkw)
    h, y2 = _tc1(xp, agg1, deg, W_self1, W_neigh1, b1.reshape(1, H1), W_neigh2)
    agg2 = _segsum(y2, pkw)
    A, Bm = _tc2(h, agg2, deg, W_self2, b2.reshape(1, H2),
                 Wp1[:H2], Wp1[H2:], bp1.reshape(1, H2))

    ep = pos_edge_index.shape[1]
    spad = NW * EPW - 2 * ep
    es = jnp.concatenate([pos_edge_index[0], neg_edge_index[0],
                          jnp.zeros((spad,), jnp.int32)]).reshape(NC, NS, NCH, C)
    ed = jnp.concatenate([pos_edge_index[1], neg_edge_index[1],
                          jnp.zeros((spad,), jnp.int32)]).reshape(NC, NS, NCH, C)
    scores = _score(A, Bm, es, ed, Wp2.reshape(DD),
                    jnp.broadcast_to(bp2, (L,)).astype(jnp.float32))
    flat = scores.reshape(-1)
    return flat[:ep], flat[ep:2 * ep]


# speedup vs baseline: 1.0374x; 1.0374x over previous
import functools

import jax
import jax.numpy as jnp
from jax import lax
from jax.experimental import pallas as pl
from jax.experimental.pallas import tpu as pltpu
from jax.experimental.pallas import tpu_sc as plsc

NC = 2
NS = 16
L = 16
NW = NC * NS
NP = 10240
DD = 128
H1 = 256
H2 = 128
C = 128
NCH = 80
EPW = C * NCH
RPT = NP // NS
PKB = 14


def _mesh():
    return plsc.VectorSubcoreMesh(core_axis_name="c", subcore_axis_name="s")


def _fill2d(ref, rows, value):
    width = ref.shape[1]

    def body(i, carry):
        r = i // (width // L)
        k = (i % (width // L)) * L
        ref[r, pl.ds(k, L)] = jnp.full((L,), value, jnp.float32)
        return carry
    lax.fori_loop(0, rows * (width // L), body, None)


def _make_deg():
    scratch = (
        pltpu.VMEM_SHARED((NP, DD), jnp.float32),
        pltpu.VMEM((NCH, C), jnp.int32),
        pltpu.VMEM((C, DD), jnp.float32),
    )

    @functools.partial(
        pl.kernel, out_type=jax.ShapeDtypeStruct((NC, NP, DD), jnp.float32),
        mesh=_mesh(), scratch_types=scratch)
    def deg_kernel(dstw, deg_out, degacc, dst_v, ones_v):
        cc = lax.axis_index("c")
        ss = lax.axis_index("s")
        base = ss * RPT
        _fill2d(ones_v, C, 0.0)
        for q in range(RPT // C):
            pltpu.sync_copy(ones_v, degacc.at[pl.ds(base + q * C, C)])
        _fill2d(ones_v, C, 1.0)
        pltpu.sync_copy(dstw.at[cc].at[ss], dst_v)
        plsc.subcore_barrier()

        def step(j, carry):
            pltpu.sync_copy(ones_v, degacc.at[dst_v.at[j]], add=True)
            return carry
        lax.fori_loop(0, NCH, step, None)
        plsc.subcore_barrier()
        pltpu.sync_copy(degacc.at[pl.ds(base, RPT)],
                        deg_out.at[cc].at[pl.ds(base, RPT)])

    return deg_kernel


_deg_pass = _make_deg()


def _make_segsum():
    scratch = (
        pltpu.VMEM_SHARED((NP, DD), jnp.float32),
        pltpu.VMEM((NCH, C), jnp.int32),
        pltpu.VMEM((2, C), jnp.int32),
        pltpu.VMEM((2, C), jnp.int32),
        pltpu.VMEM((C, DD), jnp.float32),
        pltpu.VMEM((C, DD), jnp.float32),
        pltpu.SemaphoreType.DMA,
        pltpu.SemaphoreType.DMA,
    )

    @functools.partial(
        pl.kernel, out_type=jax.ShapeDtypeStruct((NC, NP, DD), jnp.float32),
        mesh=_mesh(), scratch_types=scratch)
    def seg(table, pkw, agg_out, acc, pk_v, sidx, didx, rows0, rows1,
            sem0, sem1):
        cc = lax.axis_index("c")
        ss = lax.axis_index("s")
        base = ss * RPT

        def unpack(j, b):
            for u in range(C // L):
                v = pk_v[j, pl.ds(u * L, L)]
                sidx[b, pl.ds(u * L, L)] = jnp.bitwise_and(v, (1 << PKB) - 1)
                didx[b, pl.ds(u * L, L)] = jnp.right_shift(v, PKB)

        _fill2d(rows0, C, 0.0)
        for q in range(RPT // C):
            pltpu.sync_copy(rows0, acc.at[pl.ds(base + q * C, C)])
        pltpu.sync_copy(pkw.at[cc].at[ss], pk_v)
        plsc.subcore_barrier()

        GS = 4
        CH = C // GS

        def fire(b, rows, sem):
            for hh in range(GS):
                pltpu.async_copy(
                    table.at[sidx.at[b].at[pl.ds(hh * CH, CH)]],
                    rows.at[pl.ds(hh * CH, CH)], sem)

        def drain(b, rows, sem):
            for hh in range(GS):
                pltpu.make_async_copy(
                    table.at[sidx.at[b].at[pl.ds(hh * CH, CH)]],
                    rows.at[pl.ds(hh * CH, CH)], sem).wait()

        unpack(0, 0)
        fire(0, rows0, sem0)
        unpack(1, 1)
        fire(1, rows1, sem1)

        def step(i, carry):
            for b, rows, sem in ((0, rows0, sem0), (1, rows1, sem1)):
                j = 2 * i + b
                drain(b, rows, sem)
                pltpu.sync_copy(rows, acc.at[didx.at[b]], add=True)
                jn = lax.rem(j + 2, NCH)
                unpack(jn, b)
                fire(b, rows, sem)
            return carry
        lax.fori_loop(0, NCH // 2, step, None)
        drain(0, rows0, sem0)
        drain(1, rows1, sem1)
        plsc.subcore_barrier()
        pltpu.sync_copy(acc.at[pl.ds(base, RPT)],
                        agg_out.at[cc].at[pl.ds(base, RPT)])

    return seg


_segsum = _make_segsum()


def _make_score():
    CP = 32
    NCHP = EPW // CP
    scratch = (
        pltpu.VMEM_SHARED((NP, DD), jnp.float32),
        pltpu.VMEM((NCH, C), jnp.int32),
        pltpu.VMEM((4, CP), jnp.int32),
        pltpu.VMEM((4, CP), jnp.int32),
        pltpu.VMEM((CP, DD), jnp.float32),
        pltpu.VMEM((CP, DD), jnp.float32),
        pltpu.VMEM((CP, DD), jnp.float32),
        pltpu.VMEM((CP, DD), jnp.float32),
        pltpu.VMEM((EPW,), jnp.float32),
        pltpu.VMEM((DD,), jnp.float32),
        pltpu.VMEM((L,), jnp.float32),
        pltpu.SemaphoreType.DMA,
        pltpu.SemaphoreType.DMA,
        pltpu.SemaphoreType.DMA,
        pltpu.SemaphoreType.DMA,
    )

    @functools.partial(
        pl.kernel,
        out_type=jax.ShapeDtypeStruct((NC, NS, EPW), jnp.float32),
        mesh=_mesh(), scratch_types=scratch)
    def score(Atab, Btab, pkw, w2, bp2, out, Bsp, pk_v, sidx, didx,
              a0, a1, b0, b1, sbuf, w2v, bp2v, sA0, sA1, sB0, sB1):
        cc = lax.axis_index("c")
        ss = lax.axis_index("s")
        base = ss * RPT
        pltpu.sync_copy(pkw.at[cc].at[ss], pk_v)
        pltpu.sync_copy(w2, w2v)
        pltpu.sync_copy(bp2, bp2v)
        pltpu.sync_copy(Btab.at[pl.ds(base, RPT)], Bsp.at[pl.ds(base, RPT)])
        plsc.subcore_barrier()

        def unpack(j, slot):
            r = j // (C // CP)
            cb = lax.rem(j, C // CP) * CP
            for u in range(CP // L):
                v = pk_v[r, pl.ds(cb + u * L, L)]
                sidx[slot, pl.ds(u * L, L)] = jnp.bitwise_and(
                    v, (1 << PKB) - 1)
                didx[slot, pl.ds(u * L, L)] = jnp.right_shift(v, PKB)

        def fire(j, abuf, bbuf, sA, sB):
            slot = lax.rem(j, 4)
            pltpu.async_copy(Atab.at[sidx.at[slot]], abuf, sA)
            pltpu.async_copy(Bsp.at[didx.at[slot]], bbuf, sB)

        def drain(j, abuf, bbuf, sA, sB):
            slot = lax.rem(j, 4)
            pltpu.make_async_copy(Atab.at[sidx.at[slot]], abuf, sA).wait()
            pltpu.make_async_copy(Bsp.at[didx.at[slot]], bbuf, sB).wait()

        iota = lax.iota(jnp.int32, L)
        bp2s = bp2v[...] * (1.0 / L)
        w2regs = [w2v[pl.ds(u * L, L)] for u in range(DD // L)]
        shuf = [jnp.bitwise_xor(iota, sh) for sh in (1, 2, 4, 8)]

        unpack(0, 0)
        unpack(1, 1)
        fire(0, a0, b0, sA0, sB0)
        fire(1, a1, b1, sA1, sB1)

        def step(i, carry):
            for b, abuf, bbuf, sA, sB in ((0, a0, b0, sA0, sB0),
                                          (1, a1, b1, sA1, sB1)):
                j = 2 * i + b
                drain(j, abuf, bbuf, sA, sB)

                def group(g, carry2):
                    def edge(m, svec):
                        e = g * L + m
                        acc = bp2s
                        for u in range(DD // L):
                            va = abuf[e, pl.ds(u * L, L)]
                            vb = bbuf[e, pl.ds(u * L, L)]
                            acc = acc + jnp.maximum(va + vb, 0.0) * w2regs[u]
                        for sx in shuf:
                            acc = acc + acc.at[sx].get(
                                mode="promise_in_bounds")
                        return jnp.where(iota == m, acc, svec)
                    svec = lax.fori_loop(0, L, edge,
                                         jnp.zeros((L,), jnp.float32),
                                         unroll=4)
                    sbuf[pl.ds(j * CP + g * L, L)] = svec
                    return carry2
                lax.fori_loop(0, CP // L, group, None)
                jn = lax.rem(j + 2, NCHP)
                unpack(jn, lax.rem(jn, 4))
                fire(jn, abuf, bbuf, sA, sB)
            return carry
        lax.fori_loop(0, NCHP // 2, step, None)
        drain(0, a0, b0, sA0, sB0)
        drain(1, a1, b1, sA1, sB1)
        pltpu.sync_copy(sbuf, out.at[cc].at[ss])

    return score


_score = _make_score()

_BM = 256


def _tc1(xp, agg, deg, Ws1, Wn1, b1r, Wn2):
    def body(x_ref, agg_ref, deg_ref, ws1, wn1, b1_, wn2, h_ref, y2_ref):
        aggs = agg_ref[0] + agg_ref[1]
        dg = deg_ref[0, :, 0:1] + deg_ref[1, :, 0:1]
        mean = aggs / jnp.maximum(dg, 1.0)
        h = jnp.dot(x_ref[...], ws1[...], preferred_element_type=jnp.float32)
        h = h + jnp.dot(mean, wn1[...], preferred_element_type=jnp.float32)
        h = jnp.maximum(h + b1_[...], 0.0)
        h_ref[...] = h
        y2_ref[...] = jnp.dot(h, wn2[...], preferred_element_type=jnp.float32)

    return pl.pallas_call(
        body,
        grid=(NP // _BM,),
        in_specs=[
            pl.BlockSpec((_BM, DD), lambda i: (i, 0)),
            pl.BlockSpec((NC, _BM, DD), lambda i: (0, i, 0)),
            pl.BlockSpec((NC, _BM, DD), lambda i: (0, i, 0)),
            pl.BlockSpec((DD, H1), lambda i: (0, 0)),
            pl.BlockSpec((DD, H1), lambda i: (0, 0)),
            pl.BlockSpec((1, H1), lambda i: (0, 0)),
            pl.BlockSpec((H1, H2), lambda i: (0, 0)),
        ],
        out_specs=[
            pl.BlockSpec((_BM, H1), lambda i: (i, 0)),
            pl.BlockSpec((_BM, H2), lambda i: (i, 0)),
        ],
        out_shape=[
            jax.ShapeDtypeStruct((NP, H1), jnp.float32),
            jax.ShapeDtypeStruct((NP, H2), jnp.float32),
        ],
    )(xp, agg, deg, Ws1, Wn1, b1r, Wn2)


def _tc2(h, agg2, deg, Ws2, b2r, Wp1a, Wp1b, bp1r):
    def body(h_ref, agg_ref, deg_ref, ws2, b2_, wa, wb, bp1_, A_ref, B_ref):
        aggs = agg_ref[0] + agg_ref[1]
        dg = deg_ref[0, :, 0:1] + deg_ref[1, :, 0:1]
        mean = aggs / jnp.maximum(dg, 1.0)
        emb = jnp.dot(h_ref[...], ws2[...], preferred_element_type=jnp.float32)
        emb = emb + mean + b2_[...]
        A_ref[...] = jnp.dot(emb, wa[...], preferred_element_type=jnp.float32)
        B_ref[...] = (jnp.dot(emb, wb[...], preferred_element_type=jnp.float32)
                      + bp1_[...])

    return pl.pallas_call(
        body,
        grid=(NP // _BM,),
        in_specs=[
            pl.BlockSpec((_BM, H1), lambda i: (i, 0)),
            pl.BlockSpec((NC, _BM, H2), lambda i: (0, i, 0)),
            pl.BlockSpec((NC, _BM, DD), lambda i: (0, i, 0)),
            pl.BlockSpec((H1, H2), lambda i: (0, 0)),
            pl.BlockSpec((1, H2), lambda i: (0, 0)),
            pl.BlockSpec((H2, H2), lambda i: (0, 0)),
            pl.BlockSpec((H2, H2), lambda i: (0, 0)),
            pl.BlockSpec((1, H2), lambda i: (0, 0)),
        ],
        out_specs=[
            pl.BlockSpec((_BM, H2), lambda i: (i, 0)),
            pl.BlockSpec((_BM, H2), lambda i: (i, 0)),
        ],
        out_shape=[
            jax.ShapeDtypeStruct((NP, H2), jnp.float32),
            jax.ShapeDtypeStruct((NP, H2), jnp.float32),
        ],
    )(h, agg2, deg, Ws2, b2r, Wp1a, Wp1b, bp1r)


def kernel(x, edge_index, pos_edge_index, neg_edge_index,
           W_self1, W_neigh1, b1, W_self2, W_neigh2, b2,
           Wp1, bp1, Wp2, bp2):
    n, d = x.shape
    xp = jnp.zeros((NP, d), jnp.float32).at[:n].set(x)
    src, dst = edge_index[0], edge_index[1]
    e = src.shape[0]
    npad = NW * EPW - e
    srcf = jnp.concatenate([src, jnp.zeros((npad,), jnp.int32)])
    dstf = jnp.concatenate([dst, jnp.full((npad,), n, jnp.int32)])
    pkw = (jnp.left_shift(dstf, PKB) | srcf).reshape(NC, NS, NCH, C)
    dstw = dstf.reshape(NC, NS, NCH, C)

    deg = _deg_pass(dstw)
    agg1 = _segsum(xp, pkw)
    h, y2 = _tc1(xp, agg1, deg, W_self1, W_neigh1, b1.reshape(1, H1), W_neigh2)
    agg2 = _segsum(y2, pkw)
    A, Bm = _tc2(h, agg2, deg, W_self2, b2.reshape(1, H2),
                 Wp1[:H2], Wp1[H2:], bp1.reshape(1, H2))

    ep = pos_edge_index.shape[1]
    spad = NW * EPW - 2 * ep
    es = jnp.concatenate([pos_edge_index[0], neg_edge_index[0],
                          jnp.zeros((spad,), jnp.int32)])
    ed = jnp.concatenate([pos_edge_index[1], neg_edge_index[1],
                          jnp.zeros((spad,), jnp.int32)])
    pkw_sc = (jnp.left_shift(ed, PKB) | es).reshape(NC, NS, NCH, C)
    scores = _score(A, Bm, pkw_sc, Wp2.reshape(DD),
                    jnp.broadcast_to(bp2, (L,)).astype(jnp.float32))
    flat = scores.reshape(-1)
    return flat[:ep], flat[ep:2 * ep]
